# initial kernel scaffold (unmeasured)
import jax
import jax.numpy as jnp
from jax import lax
from jax.experimental import pallas as pl
from jax.experimental.pallas import tpu as pltpu

N_DEV = 4


def _gated_partial(x, Wg, Wu, Wd):
    m, d = x.shape
    _, h = Wg.shape
    n = Wd.shape[1]
    TH = 512
    grid = h // TH

    def body(x_ref, wg_ref, wu_ref, wd_ref, o_ref):
        j = pl.program_id(0)
        xb = x_ref[...].astype(jnp.bfloat16)
        gate = jnp.dot(
            xb, wg_ref[...].astype(jnp.bfloat16),
            preferred_element_type=jnp.float32,
        )
        up = jnp.dot(
            xb, wu_ref[...].astype(jnp.bfloat16),
            preferred_element_type=jnp.float32,
        )
        hact = (gate * up * jax.nn.sigmoid(up)).astype(jnp.bfloat16)
        part = jnp.dot(
            hact, wd_ref[...].astype(jnp.bfloat16),
            preferred_element_type=jnp.float32,
        )

        @pl.when(j == 0)
        def _():
            o_ref[...] = part

        @pl.when(j > 0)
        def _():
            o_ref[...] += part

    return pl.pallas_call(
        body,
        grid=(grid,),
        in_specs=[
            pl.BlockSpec((m, d), lambda j: (0, 0)),
            pl.BlockSpec((d, TH), lambda j: (0, j)),
            pl.BlockSpec((d, TH), lambda j: (0, j)),
            pl.BlockSpec((TH, n), lambda j: (j, 0)),
        ],
        out_specs=pl.BlockSpec((m, n), lambda j: (0, 0)),
        out_shape=jax.ShapeDtypeStruct((m, n), jnp.float32),
        compiler_params=pltpu.CompilerParams(
            dimension_semantics=("arbitrary",),
        ),
    )(x, Wg, Wu, Wd)


def _ring_allreduce(partial):
    m, n = partial.shape
    chunk = m // N_DEV
    n_hops = 2 * (N_DEV - 1)

    def body(p_ref, o_ref, comm_ref, send_sems, recv_sems):
        my = lax.axis_index("i")
        left = lax.rem(my + N_DEV - 1, N_DEV)
        right = lax.rem(my + 1, N_DEV)

        barrier_sem = pltpu.get_barrier_semaphore()
        for nbr in (left, right):
            pl.semaphore_signal(
                barrier_sem, inc=1,
                device_id=(nbr,), device_id_type=pl.DeviceIdType.MESH,
            )
        pl.semaphore_wait(barrier_sem, 2)

        def rows(c):
            return pl.ds(c * chunk, chunk)

        def hop(t):
            rdma = pltpu.make_async_remote_copy(
                src_ref=comm_ref.at[t],
                dst_ref=comm_ref.at[t + 1],
                send_sem=send_sems.at[t],
                recv_sem=recv_sems.at[t],
                device_id=(right,),
                device_id_type=pl.DeviceIdType.MESH,
            )
            rdma.start()
            rdma.wait()

        comm_ref[0] = p_ref[rows(my), :]
        for t in range(N_DEV - 1):
            hop(t)
            c = lax.rem(my + N_DEV - 1 - t, N_DEV)
            comm_ref[t + 1] += p_ref[rows(c), :]

        c_own = lax.rem(my + 1, N_DEV)
        o_ref[rows(c_own), :] = comm_ref[N_DEV - 1]

        for t in range(N_DEV - 1, n_hops):
            hop(t)
            c = lax.rem(my + N_DEV - 1 - (t - N_DEV), N_DEV)
            o_ref[rows(c), :] = comm_ref[t + 1]

    return pl.pallas_call(
        body,
        out_shape=jax.ShapeDtypeStruct((m, n), jnp.float32),
        in_specs=[pl.BlockSpec(memory_space=pltpu.VMEM)],
        out_specs=pl.BlockSpec(memory_space=pltpu.VMEM),
        scratch_shapes=[
            pltpu.VMEM((n_hops + 1, chunk, n), jnp.float32),
            pltpu.SemaphoreType.DMA((n_hops,)),
            pltpu.SemaphoreType.DMA((n_hops,)),
        ],
        compiler_params=pltpu.CompilerParams(collective_id=0),
    )(partial)


def kernel(x, Wg, Wu, Wd):
    partial = _gated_partial(x, Wg, Wu, Wd)
    return _ring_allreduce(partial)


# baseline (device time: 468977 ns/iter reference)
import jax
import jax.numpy as jnp
from jax import lax
from jax.experimental import pallas as pl
from jax.experimental.pallas import tpu as pltpu

N_DEV = 4


def _gated_partial(x, Wg, Wu, Wd):
    m, d = x.shape
    _, h = Wg.shape
    n = Wd.shape[1]
    TH = 256
    grid = h // TH

    def body(x_ref, wg_ref, wu_ref, wd_ref, o_ref):
        j = pl.program_id(0)
        xb = x_ref[...]
        gate = jnp.dot(
            xb, wg_ref[...].astype(jnp.bfloat16),
            preferred_element_type=jnp.float32,
        )
        up = jnp.dot(
            xb, wu_ref[...].astype(jnp.bfloat16),
            preferred_element_type=jnp.float32,
        )
        hact = (gate * up * jax.nn.sigmoid(up)).astype(jnp.bfloat16)
        part = jnp.dot(
            hact, wd_ref[...].astype(jnp.bfloat16),
            preferred_element_type=jnp.float32,
        )

        @pl.when(j == 0)
        def _():
            o_ref[...] = part

        @pl.when(j > 0)
        def _():
            o_ref[...] += part

    return pl.pallas_call(
        body,
        grid=(grid,),
        in_specs=[
            pl.BlockSpec((m, d), lambda j: (0, 0)),
            pl.BlockSpec((d, TH), lambda j: (0, j)),
            pl.BlockSpec((d, TH), lambda j: (0, j)),
            pl.BlockSpec((TH, n), lambda j: (j, 0)),
        ],
        out_specs=pl.BlockSpec((m, n), lambda j: (0, 0)),
        out_shape=jax.ShapeDtypeStruct((m, n), jnp.float32),
        compiler_params=pltpu.CompilerParams(
            dimension_semantics=("arbitrary",),
            vmem_limit_bytes=64 * 1024 * 1024,
        ),
    )(x.astype(jnp.bfloat16), Wg, Wu, Wd)


def _ring_allreduce(partial):
    m, n = partial.shape
    chunk = m // N_DEV
    n_hops = 2 * (N_DEV - 1)

    def body(p_ref, o_ref, comm_ref, send_sems, recv_sems):
        my = lax.axis_index("i")
        left = lax.rem(my + N_DEV - 1, N_DEV)
        right = lax.rem(my + 1, N_DEV)

        barrier_sem = pltpu.get_barrier_semaphore()
        for nbr in (left, right):
            pl.semaphore_signal(
                barrier_sem, inc=1,
                device_id=(nbr,), device_id_type=pl.DeviceIdType.MESH,
            )
        pl.semaphore_wait(barrier_sem, 2)

        def rows(c):
            return pl.ds(c * chunk, chunk)

        def hop(t):
            rdma = pltpu.make_async_remote_copy(
                src_ref=comm_ref.at[t],
                dst_ref=comm_ref.at[t + 1],
                send_sem=send_sems.at[t],
                recv_sem=recv_sems.at[t],
                device_id=(right,),
                device_id_type=pl.DeviceIdType.MESH,
            )
            rdma.start()
            rdma.wait()

        comm_ref[0] = p_ref[rows(my), :]
        for t in range(N_DEV - 1):
            hop(t)
            c = lax.rem(my + N_DEV - 1 - t, N_DEV)
            comm_ref[t + 1] += p_ref[rows(c), :]

        c_own = lax.rem(my + 1, N_DEV)
        o_ref[rows(c_own), :] = comm_ref[N_DEV - 1]

        for t in range(N_DEV - 1, n_hops):
            hop(t)
            c = lax.rem(my + N_DEV - 1 - (t - N_DEV), N_DEV)
            o_ref[rows(c), :] = comm_ref[t + 1]

    return pl.pallas_call(
        body,
        out_shape=jax.ShapeDtypeStruct((m, n), jnp.float32),
        in_specs=[pl.BlockSpec(memory_space=pltpu.VMEM)],
        out_specs=pl.BlockSpec(memory_space=pltpu.VMEM),
        scratch_shapes=[
            pltpu.VMEM((n_hops + 1, chunk, n), jnp.float32),
            pltpu.SemaphoreType.DMA((n_hops,)),
            pltpu.SemaphoreType.DMA((n_hops,)),
        ],
        compiler_params=pltpu.CompilerParams(
            collective_id=0,
            vmem_limit_bytes=64 * 1024 * 1024,
        ),
    )(partial)


def kernel(x, Wg, Wu, Wd):
    partial = _gated_partial(x, Wg, Wu, Wd)
    return _ring_allreduce(partial)


# device time: 266316 ns/iter; 1.7610x vs baseline; 1.7610x over previous
import jax
import jax.numpy as jnp
from jax import lax
from jax.experimental import pallas as pl
from jax.experimental.pallas import tpu as pltpu

N_DEV = 4


def _gated_partial(x, Wg, Wu, Wd):
    m, d = x.shape
    _, h = Wg.shape
    n = Wd.shape[1]
    TH = 256
    grid = h // TH

    def body(x_ref, wg_ref, wu_ref, wd_ref, o_ref):
        j = pl.program_id(0)
        xb = x_ref[...]
        gate = jnp.dot(
            xb, wg_ref[...].astype(jnp.bfloat16),
            preferred_element_type=jnp.float32,
        )
        up = jnp.dot(
            xb, wu_ref[...].astype(jnp.bfloat16),
            preferred_element_type=jnp.float32,
        )
        hact = (gate * up * jax.nn.sigmoid(up)).astype(jnp.bfloat16)
        part = jnp.dot(
            hact, wd_ref[...].astype(jnp.bfloat16),
            preferred_element_type=jnp.float32,
        )

        @pl.when(j == 0)
        def _():
            o_ref[...] = part

        @pl.when(j > 0)
        def _():
            o_ref[...] += part

    return pl.pallas_call(
        body,
        grid=(grid,),
        in_specs=[
            pl.BlockSpec((m, d), lambda j: (0, 0)),
            pl.BlockSpec((d, TH), lambda j: (0, j)),
            pl.BlockSpec((d, TH), lambda j: (0, j)),
            pl.BlockSpec((TH, n), lambda j: (j, 0)),
        ],
        out_specs=pl.BlockSpec((m, n), lambda j: (0, 0)),
        out_shape=jax.ShapeDtypeStruct((m, n), jnp.float32),
        compiler_params=pltpu.CompilerParams(
            dimension_semantics=("arbitrary",),
            vmem_limit_bytes=64 * 1024 * 1024,
        ),
    )(x.astype(jnp.bfloat16), Wg, Wu, Wd)


def _ring_allreduce(partial):
    m, n = partial.shape
    half = m // 2
    chunk = half // N_DEV
    n_hops = 2 * (N_DEV - 1)

    def body(p_ref, o_ref, comm_a, comm_b, sems_a, sems_b):
        my = lax.axis_index("i")
        left = lax.rem(my + N_DEV - 1, N_DEV)
        right = lax.rem(my + 1, N_DEV)

        barrier_sem = pltpu.get_barrier_semaphore()
        for nbr in (left, right):
            pl.semaphore_signal(
                barrier_sem, inc=1,
                device_id=(nbr,), device_id_type=pl.DeviceIdType.MESH,
            )
        pl.semaphore_wait(barrier_sem, 2)

        def rows_a(c):
            return pl.ds(c * chunk, chunk)

        def rows_b(c):
            return pl.ds(half + c * chunk, chunk)

        def make_hop(comm_ref, sems, t, dst):
            return pltpu.make_async_remote_copy(
                src_ref=comm_ref.at[t],
                dst_ref=comm_ref.at[t + 1],
                send_sem=sems.at[0, t],
                recv_sem=sems.at[1, t],
                device_id=(dst,),
                device_id_type=pl.DeviceIdType.MESH,
            )

        def acc(comm_ref, t, rows_sl):
            comm_ref[t + 1] = (
                comm_ref[t + 1][...].astype(jnp.float32) + p_ref[rows_sl, :]
            ).astype(jnp.bfloat16)

        comm_a[0] = p_ref[rows_a(my), :].astype(jnp.bfloat16)
        comm_b[0] = p_ref[rows_b(my), :].astype(jnp.bfloat16)

        for t in range(N_DEV - 1):
            ra = make_hop(comm_a, sems_a, t, right)
            rb = make_hop(comm_b, sems_b, t, left)
            ra.start()
            rb.start()
            ra.wait()
            acc(comm_a, t, rows_a(lax.rem(my + N_DEV - 1 - t, N_DEV)))
            rb.wait()
            acc(comm_b, t, rows_b(lax.rem(my + 1 + t, N_DEV)))

        o_ref[rows_a(lax.rem(my + 1, N_DEV)), :] = comm_a[
            N_DEV - 1
        ].astype(jnp.float32)
        o_ref[rows_b(lax.rem(my + N_DEV - 1, N_DEV)), :] = comm_b[
            N_DEV - 1
        ].astype(jnp.float32)

        for t in range(N_DEV - 1, n_hops):
            ra = make_hop(comm_a, sems_a, t, right)
            rb = make_hop(comm_b, sems_b, t, left)
            ra.start()
            rb.start()
            ta = t - N_DEV
            ra.wait()
            c = lax.rem(my + N_DEV - 1 - ta, N_DEV)
            o_ref[rows_a(c), :] = comm_a[t + 1].astype(jnp.float32)
            rb.wait()
            c = lax.rem(my + 1 + ta, N_DEV)
            o_ref[rows_b(c), :] = comm_b[t + 1].astype(jnp.float32)

    return pl.pallas_call(
        body,
        out_shape=jax.ShapeDtypeStruct((m, n), jnp.float32),
        in_specs=[pl.BlockSpec(memory_space=pltpu.VMEM)],
        out_specs=pl.BlockSpec(memory_space=pltpu.VMEM),
        scratch_shapes=[
            pltpu.VMEM((n_hops + 1, chunk, n), jnp.bfloat16),
            pltpu.VMEM((n_hops + 1, chunk, n), jnp.bfloat16),
            pltpu.SemaphoreType.DMA((2, n_hops)),
            pltpu.SemaphoreType.DMA((2, n_hops)),
        ],
        compiler_params=pltpu.CompilerParams(
            collective_id=0,
            vmem_limit_bytes=64 * 1024 * 1024,
        ),
    )(partial)


def kernel(x, Wg, Wu, Wd):
    partial = _gated_partial(x, Wg, Wu, Wd)
    return _ring_allreduce(partial)
